# Initial kernel scaffold; baseline (speedup 1.0000x reference)
import jax
import jax.numpy as jnp
from jax.experimental import pallas as pl


def kernel(subword_sequences, token_embedding):
    raise NotImplementedError("write your pallas kernel here")


# speedup vs baseline: 2.9587x; 2.9587x over previous
import functools

import jax
import jax.numpy as jnp
from jax import lax
from jax.experimental import pallas as pl
from jax.experimental.pallas import tpu as pltpu
from jax.experimental.pallas import tpu_sc as plsc

BATCH = 4096
SEQ = 50
EMBED = 128
TOTAL = BATCH * SEQ
NUM_CORES = 2
NUM_SUBCORES = 16
NW = NUM_CORES * NUM_SUBCORES
PER_W = TOTAL // NW
CHUNK = 128
NCHUNK = PER_W // CHUNK

_mesh = plsc.VectorSubcoreMesh(core_axis_name="c", subcore_axis_name="s")


@functools.partial(
    pl.kernel,
    mesh=_mesh,
    out_type=jax.ShapeDtypeStruct((TOTAL, EMBED), jnp.float32),
    scratch_types=[
        pltpu.VMEM((NCHUNK, CHUNK), jnp.int32),
        pltpu.VMEM((CHUNK, EMBED), jnp.float32),
        pltpu.SemaphoreType.DMA,
    ],
)
def _sc_gather(table_hbm, idx_hbm, out_hbm, idx_v, rows_v, gsem):
    wid = lax.axis_index("s") * NUM_CORES + lax.axis_index("c")
    base = wid * PER_W
    pltpu.sync_copy(idx_hbm.at[wid], idx_v)

    def body(j, carry):
        pltpu.async_copy(table_hbm.at[idx_v.at[j]], rows_v, gsem).wait()
        pltpu.sync_copy(rows_v, out_hbm.at[pl.ds(base + j * CHUNK, CHUNK)])
        return carry

    lax.fori_loop(0, NCHUNK, body, 0)


def kernel(subword_sequences, token_embedding):
    idx = subword_sequences.astype(jnp.int32).reshape(NW, NCHUNK, CHUNK)
    flat = _sc_gather(token_embedding, idx)
    return flat.reshape(BATCH, SEQ, EMBED)


# speedup vs baseline: 3.3287x; 1.1250x over previous
import functools

import jax
import jax.numpy as jnp
from jax import lax
from jax.experimental import pallas as pl
from jax.experimental.pallas import tpu as pltpu
from jax.experimental.pallas import tpu_sc as plsc

BATCH = 4096
SEQ = 50
EMBED = 128
TOTAL = BATCH * SEQ
NUM_CORES = 2
NUM_SUBCORES = 16
NW = NUM_CORES * NUM_SUBCORES
PER_W = TOTAL // NW
CHUNK = 128
NCHUNK = PER_W // CHUNK

_mesh = plsc.VectorSubcoreMesh(core_axis_name="c", subcore_axis_name="s")


@functools.partial(
    pl.kernel,
    mesh=_mesh,
    out_type=jax.ShapeDtypeStruct((TOTAL, EMBED), jnp.float32),
    scratch_types=[
        pltpu.VMEM((NCHUNK, CHUNK), jnp.int32),
        pltpu.VMEM((2, CHUNK, EMBED), jnp.float32),
        pltpu.SemaphoreType.DMA,
        pltpu.SemaphoreType.DMA,
        pltpu.SemaphoreType.DMA,
        pltpu.SemaphoreType.DMA,
    ],
)
def _sc_gather(table_hbm, idx_hbm, out_hbm, idx_v, rows_v, g0, g1, s0, s1):
    wid = lax.axis_index("s") * NUM_CORES + lax.axis_index("c")
    base = wid * PER_W
    gsem = (g0, g1)
    ssem = (s0, s1)
    pltpu.sync_copy(idx_hbm.at[wid], idx_v)

    def start_gather(j, b):
        pltpu.async_copy(table_hbm.at[idx_v.at[j]], rows_v.at[b], gsem[b])

    def wait_gather(j, b):
        pltpu.make_async_copy(table_hbm.at[idx_v.at[j]], rows_v.at[b],
                              gsem[b]).wait()

    def start_store(j, b):
        pltpu.async_copy(rows_v.at[b],
                         out_hbm.at[pl.ds(base + j * CHUNK, CHUNK)], ssem[b])

    def wait_store(j, b):
        pltpu.make_async_copy(rows_v.at[b],
                              out_hbm.at[pl.ds(base + j * CHUNK, CHUNK)],
                              ssem[b]).wait()

    start_gather(0, 0)

    def body(i, carry):
        j0 = i * 2
        for b in range(2):
            j = j0 + b
            nb = 1 - b

            @pl.when(j >= 1)
            def _():
                wait_store(j - 1, nb)

            @pl.when(j + 1 < NCHUNK)
            def _():
                start_gather(j + 1, nb)

            wait_gather(j, b)
            start_store(j, b)
        return carry

    lax.fori_loop(0, NCHUNK // 2, body, 0)
    wait_store(NCHUNK - 1, (NCHUNK - 1) % 2)


def kernel(subword_sequences, token_embedding):
    idx = subword_sequences.astype(jnp.int32).reshape(NW, NCHUNK, CHUNK)
    flat = _sc_gather(token_embedding, idx)
    return flat.reshape(BATCH, SEQ, EMBED)


# speedup vs baseline: 3.3506x; 1.0066x over previous
import functools

import jax
import jax.numpy as jnp
from jax import lax
from jax.experimental import pallas as pl
from jax.experimental.pallas import tpu as pltpu
from jax.experimental.pallas import tpu_sc as plsc

BATCH = 4096
SEQ = 50
EMBED = 128
TOTAL = BATCH * SEQ
NUM_CORES = 2
NUM_SUBCORES = 16
NW = NUM_CORES * NUM_SUBCORES
PER_W = TOTAL // NW
CHUNK = 128
NCHUNK = PER_W // CHUNK
NBUF = 5

_mesh = plsc.VectorSubcoreMesh(core_axis_name="c", subcore_axis_name="s")


@functools.partial(
    pl.kernel,
    mesh=_mesh,
    out_type=jax.ShapeDtypeStruct((TOTAL, EMBED), jnp.float32),
    scratch_types=[
        pltpu.VMEM((NCHUNK, CHUNK), jnp.int32),
        pltpu.VMEM((NBUF, CHUNK, EMBED), jnp.float32),
    ] + [pltpu.SemaphoreType.DMA] * (2 * NBUF),
)
def _sc_gather(table_hbm, idx_hbm, out_hbm, idx_v, rows_v, *sems):
    wid = lax.axis_index("s") * NUM_CORES + lax.axis_index("c")
    base = wid * PER_W
    gsem = sems[:NBUF]
    ssem = sems[NBUF:]
    pltpu.sync_copy(idx_hbm.at[wid], idx_v)

    def start_gather(j, b):
        pltpu.async_copy(table_hbm.at[idx_v.at[j]], rows_v.at[b], gsem[b])

    def wait_gather(j, b):
        pltpu.make_async_copy(table_hbm.at[idx_v.at[j]], rows_v.at[b],
                              gsem[b]).wait()

    def start_store(j, b):
        pltpu.async_copy(rows_v.at[b],
                         out_hbm.at[pl.ds(base + j * CHUNK, CHUNK)], ssem[b])

    def wait_store(j, b):
        pltpu.make_async_copy(rows_v.at[b],
                              out_hbm.at[pl.ds(base + j * CHUNK, CHUNK)],
                              ssem[b]).wait()

    for j in range(NBUF - 1):
        start_gather(j, j)

    def body(i, carry):
        j0 = i * NBUF
        for b in range(NBUF):
            j = j0 + b
            ahead = (b + NBUF - 1) % NBUF

            @pl.when(j + NBUF - 1 < NCHUNK)
            def _():
                @pl.when(j >= 1)
                def _():
                    wait_store(j - 1, ahead)

                start_gather(j + NBUF - 1, ahead)

            wait_gather(j, b)
            start_store(j, b)
        return carry

    lax.fori_loop(0, NCHUNK // NBUF, body, 0)
    for j in range(NCHUNK - NBUF, NCHUNK):
        wait_store(j, j % NBUF)


def kernel(subword_sequences, token_embedding):
    idx = subword_sequences.astype(jnp.int32).reshape(NW, NCHUNK, CHUNK)
    flat = _sc_gather(token_embedding, idx)
    return flat.reshape(BATCH, SEQ, EMBED)


# speedup vs baseline: 5.9490x; 1.7755x over previous
import functools

import jax
import jax.numpy as jnp
from jax import lax
from jax.experimental import pallas as pl
from jax.experimental.pallas import tpu as pltpu
from jax.experimental.pallas import tpu_sc as plsc

BATCH = 4096
SEQ = 50
EMBED = 128
NUM_CORES = 2
NUM_SUBCORES = 16
NW = NUM_CORES * NUM_SUBCORES
ROWS_W = BATCH // NW
CHB = 4
NCH = ROWS_W // CHB
NBUF = 4

_mesh = plsc.VectorSubcoreMesh(core_axis_name="c", subcore_axis_name="s")


@functools.partial(
    pl.kernel,
    mesh=_mesh,
    out_type=jax.ShapeDtypeStruct((BATCH, SEQ, EMBED), jnp.float32),
    scratch_types=[
        pltpu.VMEM((ROWS_W, SEQ), jnp.int32),
        pltpu.VMEM((NBUF, CHB, SEQ, EMBED), jnp.float32),
    ] + [pltpu.SemaphoreType.DMA] * (2 * NBUF),
)
def _sc_gather(table_hbm, idx_hbm, out_hbm, idx_v, rows_v, *sems):
    wid = lax.axis_index("s") * NUM_CORES + lax.axis_index("c")
    base = wid * ROWS_W
    gsem = sems[:NBUF]
    ssem = sems[NBUF:]
    pltpu.sync_copy(idx_hbm.at[pl.ds(base, ROWS_W)], idx_v)

    def start_gather(c, b):
        for g in range(CHB):
            pltpu.async_copy(table_hbm.at[idx_v.at[c * CHB + g]],
                             rows_v.at[b, g], gsem[b])

    def wait_gather(c, b):
        for g in range(CHB):
            pltpu.make_async_copy(table_hbm.at[idx_v.at[c * CHB + g]],
                                  rows_v.at[b, g], gsem[b]).wait()

    def start_store(c, b):
        pltpu.async_copy(rows_v.at[b],
                         out_hbm.at[pl.ds(base + c * CHB, CHB)], ssem[b])

    def wait_store(c, b):
        pltpu.make_async_copy(rows_v.at[b],
                              out_hbm.at[pl.ds(base + c * CHB, CHB)],
                              ssem[b]).wait()

    for c in range(NBUF - 1):
        start_gather(c, c)

    def body(i, carry):
        c0 = i * NBUF
        for k in range(NBUF):
            c = c0 + k
            ahead = (k + NBUF - 1) % NBUF

            @pl.when(c + NBUF - 1 < NCH)
            def _():
                @pl.when(c >= 1)
                def _():
                    wait_store(c - 1, ahead)

                start_gather(c + NBUF - 1, ahead)

            wait_gather(c, k)
            start_store(c, k)
        return carry

    lax.fori_loop(0, NCH // NBUF, body, 0)
    for c in range(NCH - NBUF, NCH):
        wait_store(c, c % NBUF)


def kernel(subword_sequences, token_embedding):
    return _sc_gather(token_embedding, subword_sequences.astype(jnp.int32))
